# Initial kernel scaffold; baseline (speedup 1.0000x reference)
import jax
import jax.numpy as jnp
from jax.experimental import pallas as pl


def kernel(x1, edge_index1, x2, edge_index2, W1, b1, W2, b2):
    raise NotImplementedError("write your pallas kernel here")


# speedup vs baseline: 9.4414x; 9.4414x over previous
import functools

import jax
import jax.numpy as jnp
from jax import lax
from jax.experimental import pallas as pl
from jax.experimental.pallas import tpu as pltpu
from jax.experimental.pallas import tpu_sc as plsc

F32 = jnp.float32
NC = 2
NS = 16
CH = 128


def _make_deg_kernel(n_pad, e_pad):
    epw = e_pad // NS
    nchunk = epw // CH
    rpt = n_pad // NS
    mesh = plsc.VectorSubcoreMesh(core_axis_name="c", subcore_axis_name="s")

    @functools.partial(
        pl.kernel,
        out_type=(jax.ShapeDtypeStruct((n_pad,), F32),
                  jax.ShapeDtypeStruct((n_pad,), F32)),
        mesh=mesh,
        scratch_types=[pltpu.VMEM((CH,), jnp.int32),
                       pltpu.VMEM((CH,), F32),
                       pltpu.VMEM((rpt,), F32),
                       pltpu.VMEM_SHARED((n_pad,), F32)],
    )
    def deg_kernel(dst1, dst2, out1, out2, idxb, ones, zbuf, acc):
        c = lax.axis_index("c")
        s = lax.axis_index("s")
        for i in range(CH // 16):
            ones[pl.ds(i * 16, 16)] = jnp.ones((16,), F32)
        for i in range(rpt // 16):
            zbuf[pl.ds(i * 16, 16)] = jnp.zeros((16,), F32)
        pltpu.sync_copy(zbuf, acc.at[pl.ds(s * rpt, rpt)])
        plsc.subcore_barrier()

        def run(dstref):
            base = s * epw

            def step(j, carry):
                pltpu.sync_copy(dstref.at[pl.ds(base + j * CH, CH)], idxb)
                pltpu.sync_copy(ones, acc.at[idxb], add=True)
                return carry

            lax.fori_loop(0, nchunk, step, 0)

        @pl.when(c == 0)
        def _():
            run(dst1)

        @pl.when(c == 1)
        def _():
            run(dst2)

        plsc.subcore_barrier()

        @pl.when(c == 0)
        def _():
            pltpu.sync_copy(acc.at[pl.ds(s * rpt, rpt)],
                            out1.at[pl.ds(s * rpt, rpt)])

        @pl.when(c == 1)
        def _():
            pltpu.sync_copy(acc.at[pl.ds(s * rpt, rpt)],
                            out2.at[pl.ds(s * rpt, rpt)])

    return deg_kernel


def _make_agg_kernel(n_pad, e_pad, d):
    nw = NC * NS
    epw = e_pad // nw
    nchunk = epw // CH
    rpt = n_pad // NS
    mesh = plsc.VectorSubcoreMesh(core_axis_name="c", subcore_axis_name="s")

    @functools.partial(
        pl.kernel,
        out_type=(jax.ShapeDtypeStruct((n_pad, d), F32),
                  jax.ShapeDtypeStruct((n_pad, d), F32)),
        mesh=mesh,
        scratch_types=[pltpu.VMEM((CH,), jnp.int32),
                       pltpu.VMEM((CH,), jnp.int32),
                       pltpu.VMEM((CH, d), F32),
                       pltpu.VMEM_SHARED((n_pad, d), F32),
                       pltpu.SemaphoreType.DMA],
    )
    def agg_kernel(y, src, dst, out0, out1, sidx, didx, rbuf, acc, sem):
        c = lax.axis_index("c")
        s = lax.axis_index("s")
        wid = c * NS + s

        def zrow(i, carry):
            for j in range(d // 16):
                rbuf[i, pl.ds(j * 16, 16)] = jnp.zeros((16,), F32)
            return carry

        lax.fori_loop(0, CH, zrow, 0)
        for k in range(rpt // CH):
            pltpu.sync_copy(rbuf, acc.at[pl.ds(s * rpt + k * CH, CH)])
        plsc.subcore_barrier()

        base = wid * epw

        def step(j, carry):
            eb = base + j * CH
            pltpu.sync_copy(src.at[pl.ds(eb, CH)], sidx)
            pltpu.sync_copy(dst.at[pl.ds(eb, CH)], didx)
            pltpu.async_copy(y.at[sidx], rbuf, sem).wait()
            pltpu.sync_copy(rbuf, acc.at[didx], add=True)
            return carry

        lax.fori_loop(0, nchunk, step, 0)
        plsc.subcore_barrier()

        @pl.when(c == 0)
        def _():
            pltpu.sync_copy(acc.at[pl.ds(s * rpt, rpt)],
                            out0.at[pl.ds(s * rpt, rpt)])

        @pl.when(c == 1)
        def _():
            pltpu.sync_copy(acc.at[pl.ds(s * rpt, rpt)],
                            out1.at[pl.ds(s * rpt, rpt)])

    return agg_kernel


_BR = 512


def _prep(x_pad, deg):
    n_pad, d = x_pad.shape
    grid = n_pad // _BR

    def body(x_ref, deg_ref, y_ref, dinv_ref):
        dv = lax.rsqrt(deg_ref[...] + 1.0)
        dinv_ref[...] = dv
        y_ref[...] = x_ref[...] * dv

    return pl.pallas_call(
        body,
        grid=(grid,),
        in_specs=[pl.BlockSpec((_BR, d), lambda j: (j, 0)),
                  pl.BlockSpec((_BR, 1), lambda j: (j, 0))],
        out_specs=[pl.BlockSpec((_BR, d), lambda j: (j, 0)),
                   pl.BlockSpec((_BR, 1), lambda j: (j, 0))],
        out_shape=[jax.ShapeDtypeStruct((n_pad, d), F32),
                   jax.ShapeDtypeStruct((n_pad, 1), F32)],
    )(x_pad, deg.reshape(n_pad, 1))


def _layer1(a0, a1, yprev, dinv, w, b):
    n_pad, d = a0.shape
    grid = n_pad // _BR

    def body(a0_ref, a1_ref, y_ref, dinv_ref, w_ref, b_ref, out_ref):
        dv = dinv_ref[...]
        agg = (a0_ref[...] + a1_ref[...] + y_ref[...]) * dv
        h = jnp.dot(agg, w_ref[...], preferred_element_type=F32) + b_ref[...]
        h = jnp.maximum(h, 0.0)
        nrm = jnp.sqrt(jnp.sum(h * h, axis=1, keepdims=True))
        h = h / jnp.maximum(nrm, 1e-12)
        out_ref[...] = h * dv

    return pl.pallas_call(
        body,
        grid=(grid,),
        in_specs=[pl.BlockSpec((_BR, d), lambda j: (j, 0)),
                  pl.BlockSpec((_BR, d), lambda j: (j, 0)),
                  pl.BlockSpec((_BR, d), lambda j: (j, 0)),
                  pl.BlockSpec((_BR, 1), lambda j: (j, 0)),
                  pl.BlockSpec((d, d), lambda j: (0, 0)),
                  pl.BlockSpec((1, d), lambda j: (0, 0))],
        out_specs=pl.BlockSpec((_BR, d), lambda j: (j, 0)),
        out_shape=jax.ShapeDtypeStruct((n_pad, d), F32),
    )(a0, a1, yprev, dinv, w, b.reshape(1, -1))


def _layer2(a0, a1, yprev, dinv, w, b, n_valid):
    n_pad, d = a0.shape
    grid = n_pad // _BR

    def body(a0_ref, a1_ref, y_ref, dinv_ref, w_ref, b_ref, h_ref, st_ref):
        j = pl.program_id(0)
        agg = (a0_ref[...] + a1_ref[...] + y_ref[...]) * dinv_ref[...]
        h = jnp.dot(agg, w_ref[...], preferred_element_type=F32) + b_ref[...]
        nrm = jnp.sqrt(jnp.sum(h * h, axis=1, keepdims=True))
        h = h / jnp.maximum(nrm, 1e-12)
        h_ref[...] = h

        @pl.when(j == 0)
        def _():
            st_ref[...] = jnp.zeros((8, d), F32)

        rows_valid = n_valid - j * _BR
        mask = lax.broadcasted_iota(jnp.int32, (_BR, d), 0) < rows_valid
        hm = jnp.where(mask, h, 0.0)
        s1 = jnp.sum(hm, axis=0, keepdims=True)
        s2 = jnp.sum(hm * hm, axis=0, keepdims=True)
        upd = jnp.concatenate([s1, s2, jnp.zeros((6, d), F32)], axis=0)
        st_ref[...] = st_ref[...] + upd

    return pl.pallas_call(
        body,
        grid=(grid,),
        in_specs=[pl.BlockSpec((_BR, d), lambda j: (j, 0)),
                  pl.BlockSpec((_BR, d), lambda j: (j, 0)),
                  pl.BlockSpec((_BR, d), lambda j: (j, 0)),
                  pl.BlockSpec((_BR, 1), lambda j: (j, 0)),
                  pl.BlockSpec((d, d), lambda j: (0, 0)),
                  pl.BlockSpec((1, d), lambda j: (0, 0))],
        out_specs=[pl.BlockSpec((_BR, d), lambda j: (j, 0)),
                   pl.BlockSpec((8, d), lambda j: (0, 0))],
        out_shape=[jax.ShapeDtypeStruct((n_pad, d), F32),
                   jax.ShapeDtypeStruct((8, d), F32)],
    )(a0, a1, yprev, dinv, w, b.reshape(1, -1))


def _standardize(h, stats, n_valid):
    n_pad, d = h.shape
    grid = n_pad // _BR
    nf = float(n_valid)

    def body(h_ref, st_ref, z_ref):
        s1 = st_ref[0:1, :]
        s2 = st_ref[1:2, :]
        mean = s1 / nf
        var = (s2 - s1 * s1 / nf) / (nf - 1.0)
        std = jnp.sqrt(jnp.maximum(var, 0.0))
        z_ref[...] = (h_ref[...] - mean) / std

    return pl.pallas_call(
        body,
        grid=(grid,),
        in_specs=[pl.BlockSpec((_BR, d), lambda j: (j, 0)),
                  pl.BlockSpec((8, d), lambda j: (0, 0))],
        out_specs=pl.BlockSpec((_BR, d), lambda j: (j, 0)),
        out_shape=jax.ShapeDtypeStruct((n_valid, d), F32),
    )(h, stats)


def kernel(x1, edge_index1, x2, edge_index2, W1, b1, W2, b2):
    n, d = x1.shape
    e = edge_index1.shape[1]
    n_pad = -((n + 1) // -(NS * CH)) * (NS * CH)
    e_pad = -(e // -(NC * NS * CH)) * (NC * NS * CH)

    deg_kernel = _make_deg_kernel(n_pad, e_pad)
    agg_kernel = _make_agg_kernel(n_pad, e_pad, d)

    def prep_edges(ei):
        ee = ei.astype(jnp.int32)
        pad = jnp.full((e_pad - e,), n, jnp.int32)
        return (jnp.concatenate([ee[0], pad]),
                jnp.concatenate([ee[1], pad]))

    src1, dst1 = prep_edges(edge_index1)
    src2, dst2 = prep_edges(edge_index2)
    xp1 = jnp.pad(x1, ((0, n_pad - n), (0, 0)))
    xp2 = jnp.pad(x2, ((0, n_pad - n), (0, 0)))

    deg1, deg2 = deg_kernel(dst1, dst2)

    def backbone(xp, src, dst, deg):
        y, dinv = _prep(xp, deg)
        a0, a1 = agg_kernel(y, src, dst)
        t = _layer1(a0, a1, y, dinv, W1, b1)
        a0, a1 = agg_kernel(t, src, dst)
        h, st = _layer2(a0, a1, t, dinv, W2, b2, n)
        return _standardize(h, st, n)

    z1 = backbone(xp1, src1, dst1, deg1)
    z2 = backbone(xp2, src2, dst2, deg2)
    return (z1, z2)
